# baseline (speedup 1.0000x reference)
import jax
import jax.numpy as jnp
from jax.experimental import pallas as pl

N = 10000


def _gat_conv(x, src, dst, W, a_src, a_dst, b, n):
    loop = jnp.arange(n, dtype=src.dtype)
    s = jnp.concatenate([src, loop])
    d = jnp.concatenate([dst, loop])
    h = x @ W
    alpha_s = h @ a_src
    alpha_d = h @ a_dst
    e = alpha_s[s] + alpha_d[d]
    e = jax.nn.leaky_relu(e, 0.2)
    emax = jax.ops.segment_max(e, d, num_segments=n)
    e = jnp.exp(e - emax[d])
    denom = jax.ops.segment_sum(e, d, num_segments=n)
    alpha = e / (denom[d] + 1e-16)
    msg = h[s] * alpha[:, None]
    out = jax.ops.segment_sum(msg, d, num_segments=n)
    return out + b


def _sigmoid_kernel(x_ref, o_ref):
    o_ref[...] = jax.nn.sigmoid(x_ref[...])


def kernel(x, edge_index, W1, a1s, a1d, b1, W2, a2s, a2d, b2, Wres, bres, Wfc, bfc):
    src, dst = edge_index[0], edge_index[1]
    residual = x
    h = _gat_conv(x, src, dst, W1, a1s, a1d, b1, N)
    h = jax.nn.relu(h)
    h = _gat_conv(h, src, dst, W2, a2s, a2d, b2, N)
    h = jax.nn.relu(h)
    res = residual @ Wres + bres
    h = h + res
    out = h @ Wfc + bfc
    return pl.pallas_call(
        _sigmoid_kernel,
        out_shape=jax.ShapeDtypeStruct(out.shape, out.dtype),
    )(out)


# speedup vs baseline: 33.7831x; 33.7831x over previous
import functools

import jax
import jax.numpy as jnp
from jax import lax
from jax.experimental import pallas as pl
from jax.experimental.pallas import tpu as pltpu
from jax.experimental.pallas import tpu_sc as plsc

_N = 10000
_D = 128
_E = 320000

_L = 16
_NSUB = 16
_NCORE = 2
_NW = _NCORE * _NSUB
_B = 128
_CH = 82
_EPW = _B * _CH
_EPAD = _NW * _EPW
_NP = 10240
_NT = 10016
_RPT = _NP // _NSUB

_ROWS = 1024


def _tc1_body(x_ref, w1_ref, a1s_ref, a1d_ref, wres_ref, bres_ref,
              h1_ref, as_ref, ad_ref, res_ref):
    xb = x_ref[...]
    h = jnp.dot(xb, w1_ref[...], preferred_element_type=jnp.float32)
    h1_ref[...] = h
    as_ref[...] = jnp.dot(h, a1s_ref[...], preferred_element_type=jnp.float32)
    ad_ref[...] = jnp.dot(h, a1d_ref[...], preferred_element_type=jnp.float32)
    res_ref[...] = (jnp.dot(xb, wres_ref[...], preferred_element_type=jnp.float32)
                    + bres_ref[...])


def _tc1(xp, W1, a1s, a1d, Wres, bres):
    g = _NP // _ROWS
    return pl.pallas_call(
        _tc1_body,
        grid=(g,),
        in_specs=[
            pl.BlockSpec((_ROWS, _D), lambda i: (i, 0)),
            pl.BlockSpec((_D, 32), lambda i: (0, 0)),
            pl.BlockSpec((32, 1), lambda i: (0, 0)),
            pl.BlockSpec((32, 1), lambda i: (0, 0)),
            pl.BlockSpec((_D, 64), lambda i: (0, 0)),
            pl.BlockSpec((1, 64), lambda i: (0, 0)),
        ],
        out_specs=[
            pl.BlockSpec((_ROWS, 32), lambda i: (i, 0)),
            pl.BlockSpec((_ROWS, 1), lambda i: (i, 0)),
            pl.BlockSpec((_ROWS, 1), lambda i: (i, 0)),
            pl.BlockSpec((_ROWS, 64), lambda i: (i, 0)),
        ],
        out_shape=[
            jax.ShapeDtypeStruct((_NP, 32), jnp.float32),
            jax.ShapeDtypeStruct((_NP, 1), jnp.float32),
            jax.ShapeDtypeStruct((_NP, 1), jnp.float32),
            jax.ShapeDtypeStruct((_NP, 64), jnp.float32),
        ],
    )(xp, W1, a1s, a1d, Wres, bres)


def _tc2_body(m0_ref, m1_ref, b1_ref, w2_ref, a2s_ref, a2d_ref,
              h2_ref, as_ref, ad_ref):
    m0 = m0_ref[...]
    m1 = m1_ref[...]
    num = m0[:, :32] + m1[:, :32]
    den = m0[:, 32:33] + m1[:, 32:33] + 1e-16
    z = jnp.maximum(num / den + b1_ref[...], 0.0)
    h2 = jnp.dot(z, w2_ref[...], preferred_element_type=jnp.float32)
    h2_ref[...] = h2
    as_ref[...] = jnp.dot(h2, a2s_ref[...], preferred_element_type=jnp.float32)
    ad_ref[...] = jnp.dot(h2, a2d_ref[...], preferred_element_type=jnp.float32)


def _tc2(m0, m1, b1, W2, a2s, a2d):
    g = _NP // _ROWS
    return pl.pallas_call(
        _tc2_body,
        grid=(g,),
        in_specs=[
            pl.BlockSpec((_ROWS, 48), lambda i: (i, 0)),
            pl.BlockSpec((_ROWS, 48), lambda i: (i, 0)),
            pl.BlockSpec((1, 32), lambda i: (0, 0)),
            pl.BlockSpec((32, 64), lambda i: (0, 0)),
            pl.BlockSpec((64, 1), lambda i: (0, 0)),
            pl.BlockSpec((64, 1), lambda i: (0, 0)),
        ],
        out_specs=[
            pl.BlockSpec((_ROWS, 64), lambda i: (i, 0)),
            pl.BlockSpec((_ROWS, 1), lambda i: (i, 0)),
            pl.BlockSpec((_ROWS, 1), lambda i: (i, 0)),
        ],
        out_shape=[
            jax.ShapeDtypeStruct((_NP, 64), jnp.float32),
            jax.ShapeDtypeStruct((_NP, 1), jnp.float32),
            jax.ShapeDtypeStruct((_NP, 1), jnp.float32),
        ],
    )(m0, m1, b1, W2, a2s, a2d)


def _tc3_body(m0_ref, m1_ref, b2_ref, res_ref, wfc_ref, bfc_ref, o_ref):
    m0 = m0_ref[...]
    m1 = m1_ref[...]
    num = m0[:, :64] + m1[:, :64]
    den = m0[:, 64:65] + m1[:, 64:65] + 1e-16
    z = jnp.maximum(num / den + b2_ref[...], 0.0)
    t = z + res_ref[...]
    o_ref[...] = jax.nn.sigmoid(
        jnp.dot(t, wfc_ref[...], preferred_element_type=jnp.float32) + bfc_ref[...])


def _tc3(m0, m1, b2, res, Wfc, bfc):
    g = _NP // _ROWS
    return pl.pallas_call(
        _tc3_body,
        grid=(g,),
        in_specs=[
            pl.BlockSpec((_ROWS, 80), lambda i: (i, 0)),
            pl.BlockSpec((_ROWS, 80), lambda i: (i, 0)),
            pl.BlockSpec((1, 64), lambda i: (0, 0)),
            pl.BlockSpec((_ROWS, 64), lambda i: (i, 0)),
            pl.BlockSpec((64, 1), lambda i: (0, 0)),
            pl.BlockSpec((1, 1), lambda i: (0, 0)),
        ],
        out_specs=pl.BlockSpec((_ROWS, 1), lambda i: (i, 0)),
        out_shape=jax.ShapeDtypeStruct((_NP, 1), jnp.float32),
    )(m0, m1, b2, res, Wfc, bfc)


def _make_sc_edge(F):
    Fo = F + _L
    mesh = plsc.VectorSubcoreMesh(core_axis_name="c", subcore_axis_name="s")

    @functools.partial(
        pl.kernel,
        out_type=jax.ShapeDtypeStruct((_NCORE, _NP, Fo), jnp.float32),
        mesh=mesh,
        compiler_params=pltpu.CompilerParams(needs_layout_passes=False,
                                             use_tc_tiling_on_sc=False),
        scratch_types=[
            pltpu.VMEM((_CH, _B), jnp.int32),
            pltpu.VMEM((_CH, _B), jnp.int32),
            pltpu.VMEM((_NT,), jnp.float32),
            pltpu.VMEM((_NT,), jnp.float32),
            pltpu.VMEM((_B, F), jnp.float32),
            pltpu.VMEM((_B, Fo), jnp.float32),
            pltpu.VMEM((_B,), jnp.float32),
            pltpu.VMEM_SHARED((_NP, Fo), jnp.float32),
            pltpu.SemaphoreType.DMA,
        ],
    )
    def sc_edge(src3, dst3, as_t, ad_t, h, zrows, out,
                src_b, dst_b, as_b, ad_b, gbuf, sbuf, pbuf, acc, sem):
        c = lax.axis_index("c")
        s = lax.axis_index("s")
        wid = c * _NSUB + s
        pltpu.sync_copy(src3.at[wid], src_b)
        pltpu.sync_copy(dst3.at[wid], dst_b)
        pltpu.sync_copy(as_t, as_b)
        pltpu.sync_copy(ad_t, ad_b)
        pltpu.sync_copy(zrows, acc.at[pl.ds(s * _RPT, _RPT)])
        plsc.subcore_barrier()

        def chunk_body(j, carry):
            cp = pltpu.async_copy(h.at[src_b.at[j]], gbuf, sem)
            for g in range(_B // _L):
                sv = src_b[j, pl.ds(g * _L, _L)]
                dv = dst_b[j, pl.ds(g * _L, _L)]
                u = plsc.load_gather(as_b, [sv]) + plsc.load_gather(ad_b, [dv])
                pbuf[pl.ds(g * _L, _L)] = jnp.exp(jnp.maximum(u, 0.2 * u))
            cp.wait()
            for g in range(_B // _L):
                for k in range(_L):
                    e = g * _L + k
                    pk = plsc.load_gather(pbuf, [jnp.full((_L,), e, jnp.int32)])
                    for q in range(F // _L):
                        sbuf[e, pl.ds(q * _L, _L)] = gbuf[e, pl.ds(q * _L, _L)] * pk
                    sbuf[e, pl.ds(F, _L)] = pk
            pltpu.sync_copy(sbuf, acc.at[dst_b.at[j]], add=True)
            return carry

        lax.fori_loop(0, _CH, chunk_body, 0)
        plsc.subcore_barrier()
        pltpu.sync_copy(acc.at[pl.ds(s * _RPT, _RPT)],
                        out.at[c, pl.ds(s * _RPT, _RPT)])

    return sc_edge


_sc_edge_32 = _make_sc_edge(32)
_sc_edge_64 = _make_sc_edge(64)


def kernel(x, edge_index, W1, a1s, a1d, b1, W2, a2s, a2d, b2, Wres, bres, Wfc, bfc):
    f32 = jnp.float32
    xp = jnp.pad(x.astype(f32), ((0, _NP - _N), (0, 0)))

    loop = jnp.arange(_N, dtype=jnp.int32)
    npad = _EPAD - _E - _N
    src = jnp.concatenate([edge_index[0].astype(jnp.int32), loop,
                           jnp.zeros((npad,), jnp.int32)]).reshape(_NW, _CH, _B)
    dst = jnp.concatenate([edge_index[1].astype(jnp.int32), loop,
                           jnp.full((npad,), _N, jnp.int32)]).reshape(_NW, _CH, _B)

    z48 = jnp.zeros((_RPT, 48), f32)
    z80 = jnp.zeros((_RPT, 80), f32)

    h1, as1, ad1, res = _tc1(xp, W1, a1s.reshape(32, 1), a1d.reshape(32, 1),
                             Wres, bres.reshape(1, 64))
    part1 = _sc_edge_32(src, dst, as1[:_NT, 0], ad1[:_NT, 0], h1, z48)
    h2, as2, ad2 = _tc2(part1[0], part1[1], b1.reshape(1, 32), W2,
                        a2s.reshape(64, 1), a2d.reshape(64, 1))
    part2 = _sc_edge_64(src, dst, as2[:_NT, 0], ad2[:_NT, 0], h2, z80)
    out = _tc3(part2[0], part2[1], b2.reshape(1, 64), res, Wfc, bfc.reshape(1, 1))
    return out[:_N]


# speedup vs baseline: 34.3118x; 1.0156x over previous
import functools

import jax
import jax.numpy as jnp
from jax import lax
from jax.experimental import pallas as pl
from jax.experimental.pallas import tpu as pltpu
from jax.experimental.pallas import tpu_sc as plsc

_N = 10000
_D = 128
_E = 320000

_L = 16
_NSUB = 16
_NCORE = 2
_NW = _NCORE * _NSUB
_B = 128
_CH = 82
_EPW = _B * _CH
_EPAD = _NW * _EPW
_NP = 10240
_NT = 10016
_RPT = _NP // _NSUB

_ROWS = 1024


def _tc1_body(x_ref, w1_ref, a1s_ref, a1d_ref, wres_ref, bres_ref,
              h1_ref, as_ref, ad_ref, res_ref):
    xb = x_ref[...]
    h = jnp.dot(xb, w1_ref[...], preferred_element_type=jnp.float32)
    h1_ref[...] = h
    as_ref[...] = jnp.dot(h, a1s_ref[...], preferred_element_type=jnp.float32)
    ad_ref[...] = jnp.dot(h, a1d_ref[...], preferred_element_type=jnp.float32)
    res_ref[...] = (jnp.dot(xb, wres_ref[...], preferred_element_type=jnp.float32)
                    + bres_ref[...])


def _tc1(xp, W1, a1s, a1d, Wres, bres):
    g = _NP // _ROWS
    return pl.pallas_call(
        _tc1_body,
        grid=(g,),
        in_specs=[
            pl.BlockSpec((_ROWS, _D), lambda i: (i, 0)),
            pl.BlockSpec((_D, 32), lambda i: (0, 0)),
            pl.BlockSpec((32, 1), lambda i: (0, 0)),
            pl.BlockSpec((32, 1), lambda i: (0, 0)),
            pl.BlockSpec((_D, 64), lambda i: (0, 0)),
            pl.BlockSpec((1, 64), lambda i: (0, 0)),
        ],
        out_specs=[
            pl.BlockSpec((_ROWS, 32), lambda i: (i, 0)),
            pl.BlockSpec((_ROWS, 1), lambda i: (i, 0)),
            pl.BlockSpec((_ROWS, 1), lambda i: (i, 0)),
            pl.BlockSpec((_ROWS, 64), lambda i: (i, 0)),
        ],
        out_shape=[
            jax.ShapeDtypeStruct((_NP, 32), jnp.float32),
            jax.ShapeDtypeStruct((_NP, 1), jnp.float32),
            jax.ShapeDtypeStruct((_NP, 1), jnp.float32),
            jax.ShapeDtypeStruct((_NP, 64), jnp.float32),
        ],
    )(xp, W1, a1s, a1d, Wres, bres)


def _tc2_body(m0_ref, m1_ref, b1_ref, w2_ref, a2s_ref, a2d_ref,
              h2_ref, as_ref, ad_ref):
    m0 = m0_ref[...]
    m1 = m1_ref[...]
    num = m0[:, :32] + m1[:, :32]
    den = m0[:, 32:33] + m1[:, 32:33] + 1e-16
    z = jnp.maximum(num / den + b1_ref[...], 0.0)
    h2 = jnp.dot(z, w2_ref[...], preferred_element_type=jnp.float32)
    h2_ref[...] = h2
    as_ref[...] = jnp.dot(h2, a2s_ref[...], preferred_element_type=jnp.float32)
    ad_ref[...] = jnp.dot(h2, a2d_ref[...], preferred_element_type=jnp.float32)


def _tc2(m0, m1, b1, W2, a2s, a2d):
    g = _NP // _ROWS
    return pl.pallas_call(
        _tc2_body,
        grid=(g,),
        in_specs=[
            pl.BlockSpec((_ROWS, 48), lambda i: (i, 0)),
            pl.BlockSpec((_ROWS, 48), lambda i: (i, 0)),
            pl.BlockSpec((1, 32), lambda i: (0, 0)),
            pl.BlockSpec((32, 64), lambda i: (0, 0)),
            pl.BlockSpec((64, 1), lambda i: (0, 0)),
            pl.BlockSpec((64, 1), lambda i: (0, 0)),
        ],
        out_specs=[
            pl.BlockSpec((_ROWS, 64), lambda i: (i, 0)),
            pl.BlockSpec((_ROWS, 1), lambda i: (i, 0)),
            pl.BlockSpec((_ROWS, 1), lambda i: (i, 0)),
        ],
        out_shape=[
            jax.ShapeDtypeStruct((_NP, 64), jnp.float32),
            jax.ShapeDtypeStruct((_NP, 1), jnp.float32),
            jax.ShapeDtypeStruct((_NP, 1), jnp.float32),
        ],
    )(m0, m1, b1, W2, a2s, a2d)


def _tc3_body(m0_ref, m1_ref, b2_ref, res_ref, wfc_ref, bfc_ref, o_ref):
    m0 = m0_ref[...]
    m1 = m1_ref[...]
    num = m0[:, :64] + m1[:, :64]
    den = m0[:, 64:65] + m1[:, 64:65] + 1e-16
    z = jnp.maximum(num / den + b2_ref[...], 0.0)
    t = z + res_ref[...]
    o_ref[...] = jax.nn.sigmoid(
        jnp.dot(t, wfc_ref[...], preferred_element_type=jnp.float32) + bfc_ref[...])


def _tc3(m0, m1, b2, res, Wfc, bfc):
    g = _NP // _ROWS
    return pl.pallas_call(
        _tc3_body,
        grid=(g,),
        in_specs=[
            pl.BlockSpec((_ROWS, 80), lambda i: (i, 0)),
            pl.BlockSpec((_ROWS, 80), lambda i: (i, 0)),
            pl.BlockSpec((1, 64), lambda i: (0, 0)),
            pl.BlockSpec((_ROWS, 64), lambda i: (i, 0)),
            pl.BlockSpec((64, 1), lambda i: (0, 0)),
            pl.BlockSpec((1, 1), lambda i: (0, 0)),
        ],
        out_specs=pl.BlockSpec((_ROWS, 1), lambda i: (i, 0)),
        out_shape=jax.ShapeDtypeStruct((_NP, 1), jnp.float32),
    )(m0, m1, b2, res, Wfc, bfc)


def _make_sc_edge(F):
    Fo = F + _L
    mesh = plsc.VectorSubcoreMesh(core_axis_name="c", subcore_axis_name="s")

    @functools.partial(
        pl.kernel,
        out_type=jax.ShapeDtypeStruct((_NCORE, _NP, Fo), jnp.float32),
        mesh=mesh,
        compiler_params=pltpu.CompilerParams(needs_layout_passes=False,
                                             use_tc_tiling_on_sc=False),
        scratch_types=[
            pltpu.VMEM((_CH, _B), jnp.int32),
            pltpu.VMEM((_CH, _B), jnp.int32),
            pltpu.VMEM((_NT,), jnp.float32),
            pltpu.VMEM((_NT,), jnp.float32),
            pltpu.VMEM((_B, F), jnp.float32),
            pltpu.VMEM((_B, F), jnp.float32),
            pltpu.VMEM((_B, Fo), jnp.float32),
            pltpu.VMEM((_B, Fo), jnp.float32),
            pltpu.VMEM((_B,), jnp.float32),
            pltpu.VMEM_SHARED((_NP, Fo), jnp.float32),
            pltpu.SemaphoreType.DMA,
            pltpu.SemaphoreType.DMA,
            pltpu.SemaphoreType.DMA,
            pltpu.SemaphoreType.DMA,
        ],
    )
    def sc_edge(src3, dst3, as_t, ad_t, h, zrows, out,
                src_b, dst_b, as_b, ad_b, gbuf0, gbuf1, sbuf0, sbuf1, pbuf,
                acc, sem_g0, sem_g1, sem_s0, sem_s1):
        c = lax.axis_index("c")
        s = lax.axis_index("s")
        wid = c * _NSUB + s
        pltpu.sync_copy(src3.at[wid], src_b)
        pltpu.sync_copy(dst3.at[wid], dst_b)
        pltpu.sync_copy(as_t, as_b)
        pltpu.sync_copy(ad_t, ad_b)
        pltpu.sync_copy(zrows, acc.at[pl.ds(s * _RPT, _RPT)])
        plsc.subcore_barrier()

        def compute_p(j):
            for g in range(_B // _L):
                sv = src_b[j, pl.ds(g * _L, _L)]
                dv = dst_b[j, pl.ds(g * _L, _L)]
                u = plsc.load_gather(as_b, [sv]) + plsc.load_gather(ad_b, [dv])
                pbuf[pl.ds(g * _L, _L)] = jnp.exp(jnp.maximum(u, 0.2 * u))

        def scale(gb, sb):
            for g in range(_B // _L):
                for k in range(_L):
                    e = g * _L + k
                    pk = plsc.load_gather(pbuf, [jnp.full((_L,), e, jnp.int32)])
                    for q in range(F // _L):
                        sb[e, pl.ds(q * _L, _L)] = gb[e, pl.ds(q * _L, _L)] * pk
                    sb[e, pl.ds(F, _L)] = pk

        _TH = _CH // 2

        def body(t, carry):
            j0 = 2 * t
            j1 = j0 + 1
            cp0 = pltpu.async_copy(h.at[src_b.at[j0]], gbuf0, sem_g0)
            cp1 = pltpu.async_copy(h.at[src_b.at[j1]], gbuf1, sem_g1)
            compute_p(j0)
            cp0.wait()
            scale(gbuf0, sbuf0)
            cs0 = pltpu.async_copy(sbuf0, acc.at[dst_b.at[j0]], sem_s0, add=True)
            compute_p(j1)
            cp1.wait()
            scale(gbuf1, sbuf1)
            cs1 = pltpu.async_copy(sbuf1, acc.at[dst_b.at[j1]], sem_s1, add=True)
            cs0.wait()
            cs1.wait()
            return carry

        lax.fori_loop(0, _TH, body, 0)
        plsc.subcore_barrier()
        pltpu.sync_copy(acc.at[pl.ds(s * _RPT, _RPT)],
                        out.at[c, pl.ds(s * _RPT, _RPT)])

    return sc_edge


_sc_edge_32 = _make_sc_edge(32)
_sc_edge_64 = _make_sc_edge(64)


def kernel(x, edge_index, W1, a1s, a1d, b1, W2, a2s, a2d, b2, Wres, bres, Wfc, bfc):
    f32 = jnp.float32
    xp = jnp.pad(x.astype(f32), ((0, _NP - _N), (0, 0)))

    loop = jnp.arange(_N, dtype=jnp.int32)
    npad = _EPAD - _E - _N
    src = jnp.concatenate([edge_index[0].astype(jnp.int32), loop,
                           jnp.zeros((npad,), jnp.int32)]).reshape(_NW, _CH, _B)
    dst = jnp.concatenate([edge_index[1].astype(jnp.int32), loop,
                           jnp.full((npad,), _N, jnp.int32)]).reshape(_NW, _CH, _B)

    z48 = jnp.zeros((_RPT, 48), f32)
    z80 = jnp.zeros((_RPT, 80), f32)

    h1, as1, ad1, res = _tc1(xp, W1, a1s.reshape(32, 1), a1d.reshape(32, 1),
                             Wres, bres.reshape(1, 64))
    part1 = _sc_edge_32(src, dst, as1[:_NT, 0], ad1[:_NT, 0], h1, z48)
    h2, as2, ad2 = _tc2(part1[0], part1[1], b1.reshape(1, 32), W2,
                        a2s.reshape(64, 1), a2d.reshape(64, 1))
    part2 = _sc_edge_64(src, dst, as2[:_NT, 0], ad2[:_NT, 0], h2, z80)
    out = _tc3(part2[0], part2[1], b2.reshape(1, 64), res, Wfc, bfc.reshape(1, 1))
    return out[:_N]
